# baseline (speedup 1.0000x reference)
import jax
import jax.numpy as jnp
from jax import lax
from jax.experimental import pallas as pl
from jax.experimental.pallas import tpu as pltpu
from jax.experimental.pallas import tpu_sc as plsc

_N = 2048
_NSPEC = 4
_CUTOFF = 1.0
_NW = 32
_LN2 = 0.6931471805599453

_SC_ROWS = 512
_SC_RPW = _SC_ROWS // _NW
_TC_BLOCK = 256

_LOG2_COEFFS = (
    -3.0283174810372375, 6.065830143177264, -5.2641104770701075,
    3.218832837050299, -1.2342631730323361, 0.26685882285942003,
    -0.024825606614202734,
)


def _sc_body(xs_h, ys_h, zs_h, sp_h, t3_h, ta_h, te_h, tf_h,
             fx_h, fy_h, fz_h, e_h,
             xs_v, ys_v, zs_v, sp_v, t3_v, ta_v, te_v, tf_v,
             fxv, fyv, fzv, ev):
    wid = lax.axis_index("s") * 2 + lax.axis_index("c")
    pltpu.sync_copy(xs_h, xs_v)
    pltpu.sync_copy(ys_h, ys_v)
    pltpu.sync_copy(zs_h, zs_v)
    pltpu.sync_copy(sp_h, sp_v)
    pltpu.sync_copy(t3_h, t3_v)
    pltpu.sync_copy(ta_h, ta_v)
    pltpu.sync_copy(te_h, te_v)
    pltpu.sync_copy(tf_h, tf_v)

    lanes = lax.broadcasted_iota(jnp.int32, (16,), 0)
    zero16 = jnp.zeros((16,), jnp.float32)

    def rc_body(rc, e_carry):
        base = wid * _SC_RPW + rc * 16
        xi = xs_v[pl.ds(base, 16)]
        yi = ys_v[pl.ds(base, 16)]
        zi = zs_v[pl.ds(base, 16)]
        si4 = sp_v[pl.ds(base, 16)] * 4
        iv = base + lanes

        @plsc.parallel_loop(0, _N, carry=(zero16, zero16, zero16, e_carry),
                            unroll=4)
        def j_loop(j, carry):
            fx, fy, fz, e = carry
            jf = jnp.full((16,), j, jnp.int32)
            xj = plsc.load_gather(xs_v, [jf])
            yj = plsc.load_gather(ys_v, [jf])
            zj = plsc.load_gather(zs_v, [jf])
            sj = plsc.load_gather(sp_v, [jf])
            idx = si4 + sj
            t3g = plsc.load_gather(t3_v, [idx])
            tag = plsc.load_gather(ta_v, [idx])
            teg = plsc.load_gather(te_v, [idx])
            tfg = plsc.load_gather(tf_v, [idx])
            dx = xj - xi
            dy = yj - yi
            dz = zj - zi
            d2 = dx * dx + dy * dy + dz * dz
            y0 = plsc.bitcast(
                jnp.int32(0x5F3759DF) - (plsc.bitcast(d2, jnp.int32) >> 1),
                jnp.float32)
            hd = 0.5 * d2
            y0 = y0 * (1.5 - hd * y0 * y0)
            y0 = y0 * (1.5 - hd * y0 * y0)
            r = d2 * y0
            xr = 1.0 - r * t3g
            mask = (jf != iv) & (r < _CUTOFF) & (xr > 0.0)
            xq = jnp.where(mask, xr, 1.0)
            xb = plsc.bitcast(xq, jnp.int32)
            ef = ((xb >> 23) - 127).astype(jnp.float32)
            mb = plsc.bitcast((xb & 0x007FFFFF) | 0x3F800000, jnp.float32)
            poly = jnp.full((16,), _LOG2_COEFFS[6], jnp.float32)
            for c in _LOG2_COEFFS[5::-1]:
                poly = poly * mb + c
            lg2 = poly + ef
            p1 = jnp.exp(tag * lg2)
            e = e + jnp.where(mask, teg * p1 * xq, 0.0)
            g = jnp.where(mask, (tfg * p1) * y0, 0.0)
            return fx + g * dx, fy + g * dy, fz + g * dz, e

        fx, fy, fz, e2 = j_loop
        fxv[pl.ds(rc * 16, 16)] = fx
        fyv[pl.ds(rc * 16, 16)] = fy
        fzv[pl.ds(rc * 16, 16)] = fz
        return e2

    e_final = lax.fori_loop(0, _SC_RPW // 16, rc_body, zero16)
    ev[...] = e_final
    base = wid * _SC_RPW
    pltpu.sync_copy(fxv, fx_h.at[pl.ds(base, _SC_RPW)])
    pltpu.sync_copy(fyv, fy_h.at[pl.ds(base, _SC_RPW)])
    pltpu.sync_copy(fzv, fz_h.at[pl.ds(base, _SC_RPW)])
    pltpu.sync_copy(ev, e_h.at[pl.ds(wid * 16, 16)])


def _sc_call(xs, ys, zs, spec, t3_t, ta_t, te_t, tf_t):
    mesh = plsc.VectorSubcoreMesh(core_axis_name="c", subcore_axis_name="s")
    f32 = jnp.float32
    run = pl.kernel(
        _sc_body,
        mesh=mesh,
        compiler_params=pltpu.CompilerParams(needs_layout_passes=False,
                                             skip_device_barrier=True),
        out_type=[
            jax.ShapeDtypeStruct((_SC_ROWS,), f32),
            jax.ShapeDtypeStruct((_SC_ROWS,), f32),
            jax.ShapeDtypeStruct((_SC_ROWS,), f32),
            jax.ShapeDtypeStruct((_NW * 16,), f32),
        ],
        scratch_types=[
            pltpu.VMEM((_N,), f32),
            pltpu.VMEM((_N,), f32),
            pltpu.VMEM((_N,), f32),
            pltpu.VMEM((_N,), jnp.int32),
            pltpu.VMEM((16,), f32),
            pltpu.VMEM((16,), f32),
            pltpu.VMEM((16,), f32),
            pltpu.VMEM((16,), f32),
            pltpu.VMEM((_SC_RPW,), f32),
            pltpu.VMEM((_SC_RPW,), f32),
            pltpu.VMEM((_SC_RPW,), f32),
            pltpu.VMEM((16,), f32),
        ],
    )
    return run(xs, ys, zs, spec, t3_t, ta_t, te_t, tf_t)


def _tc_body(pos_ref, posT_ref, sc_ref, sr_ref, posf_ref, t3_ref, ta_ref,
             te_ref, tf_ref, e_ref, f_ref):
    pid = pl.program_id(0)
    n = posT_ref.shape[1]
    blk = pos_ref.shape[0]

    px = posT_ref[0:1, :]
    py = posT_ref[1:2, :]
    pz = posT_ref[2:3, :]
    cx = pos_ref[:, 0:1]
    cy = pos_ref[:, 1:2]
    cz = pos_ref[:, 2:3]
    dx = px - cx
    dy = py - cy
    dz = pz - cz
    d2 = dx * dx + dy * dy + dz * dz
    rinv = jax.lax.rsqrt(d2)
    r = d2 * rinv

    oh_i = (sc_ref[...] == jax.lax.broadcasted_iota(
        jnp.int32, (blk, _NSPEC), 1)).astype(jnp.float32)
    oh_j = (sr_ref[...] == jax.lax.broadcasted_iota(
        jnp.int32, (_NSPEC, n), 0)).astype(jnp.float32)

    def pair_field(t_ref):
        return jnp.dot(
            jnp.dot(oh_i, t_ref[...], preferred_element_type=jnp.float32),
            oh_j, preferred_element_type=jnp.float32)

    t3 = pair_field(t3_ref)
    ta = pair_field(ta_ref)
    te = pair_field(te_ref)
    tf = pair_field(tf_ref)

    row = _SC_ROWS + pid * blk + jax.lax.broadcasted_iota(
        jnp.int32, (blk, n), 0)
    col = jax.lax.broadcasted_iota(jnp.int32, (blk, n), 1)
    xr = 1.0 - r * t3
    mask = (row != col) & (r < _CUTOFF) & (xr > 0.0)

    xq = jnp.where(mask, xr, 1.0)
    p1 = jnp.exp(ta * jnp.log(xq))
    energies = jnp.where(mask, te * (p1 * xq), 0.0)
    g = jnp.where(mask, (tf * p1) * rinv, 0.0)

    s = jnp.sum(g, axis=1, keepdims=True)
    f_ref[...] = (jnp.dot(g, posf_ref[...], preferred_element_type=jnp.float32)
                  - s * pos_ref[...])

    @pl.when(pid == 0)
    def _():
        e_ref[...] = jnp.zeros((1, 1), jnp.float32)

    e_ref[...] += 0.5 * jnp.sum(energies, keepdims=True)


def _tc_call(positions, species):
    n = positions.shape[0]
    nrows = n - _SC_ROWS
    posT = positions.T
    spec_col = species.reshape(n, 1).astype(jnp.int32)
    spec_row = species.reshape(1, n).astype(jnp.int32)
    row0 = _SC_ROWS // _TC_BLOCK

    def run(t3_m, ta_m, te_m, tf_m):
        return pl.pallas_call(
            _tc_body,
            grid=(nrows // _TC_BLOCK,),
            compiler_params=pltpu.CompilerParams(skip_device_barrier=True),
            in_specs=[
                pl.BlockSpec((_TC_BLOCK, 3), lambda i: (row0 + i, 0)),
                pl.BlockSpec((3, n), lambda i: (0, 0)),
                pl.BlockSpec((_TC_BLOCK, 1), lambda i: (row0 + i, 0)),
                pl.BlockSpec((1, n), lambda i: (0, 0)),
                pl.BlockSpec((n, 3), lambda i: (0, 0)),
                pl.BlockSpec((_NSPEC, _NSPEC), lambda i: (0, 0)),
                pl.BlockSpec((_NSPEC, _NSPEC), lambda i: (0, 0)),
                pl.BlockSpec((_NSPEC, _NSPEC), lambda i: (0, 0)),
                pl.BlockSpec((_NSPEC, _NSPEC), lambda i: (0, 0)),
            ],
            out_specs=[
                pl.BlockSpec((1, 1), lambda i: (0, 0)),
                pl.BlockSpec((_TC_BLOCK, 3), lambda i: (i, 0)),
            ],
            out_shape=[
                jax.ShapeDtypeStruct((1, 1), jnp.float32),
                jax.ShapeDtypeStruct((nrows, 3), jnp.float32),
            ],
        )(positions, posT, spec_col, spec_row, positions,
          t3_m, ta_m, te_m, tf_m)

    return run


def kernel(positions, species, sigma_matrix, epsilon_matrix, alpha_matrix):
    xs = positions[:, 0]
    ys = positions[:, 1]
    zs = positions[:, 2]
    spec = species.astype(jnp.int32)
    t3_t = (1.0 / sigma_matrix).reshape(-1)
    ta_t = ((alpha_matrix - 1.0) * _LN2).reshape(-1)
    te_t = (epsilon_matrix / alpha_matrix).reshape(-1)
    tf_t = (-epsilon_matrix / sigma_matrix).reshape(-1)
    fx, fy, fz, e_part = _sc_call(xs, ys, zs, spec, t3_t, ta_t, te_t, tf_t)
    e_tc, f_tc = _tc_call(positions, species)(
        1.0 / sigma_matrix, alpha_matrix - 1.0,
        epsilon_matrix / alpha_matrix, -epsilon_matrix / sigma_matrix)
    forces = jnp.concatenate(
        [jnp.stack([fx, fy, fz], axis=1), f_tc], axis=0)
    energy = 0.5 * jnp.sum(e_part) + e_tc[0, 0]
    return energy, forces


# speedup vs baseline: 1.0058x; 1.0058x over previous
import jax
import jax.numpy as jnp
from jax import lax
from jax.experimental import pallas as pl
from jax.experimental.pallas import tpu as pltpu
from jax.experimental.pallas import tpu_sc as plsc

_N = 2048
_NSPEC = 4
_CUTOFF = 1.0
_NW = 32
_LN2 = 0.6931471805599453

_SC_ROWS = 256
_SC_CHUNKS = _SC_ROWS // 16
_SC_SPLIT = _NW // _SC_CHUNKS
_SC_JSPAN = _N // _SC_SPLIT
_TC_BLOCK = 256

_LOG2_COEFFS = (
    -3.0283174810372375, 6.065830143177264, -5.2641104770701075,
    3.218832837050299, -1.2342631730323361, 0.26685882285942003,
    -0.024825606614202734,
)


def _sc_body(xs_h, ys_h, zs_h, sp_h, t3_h, ta_h, te_h, tf_h,
             fx_h, fy_h, fz_h, e_h,
             xs_v, ys_v, zs_v, sp_v, t3_v, ta_v, te_v, tf_v,
             fxv, fyv, fzv, ev):
    wid = lax.axis_index("s") * 2 + lax.axis_index("c")
    pltpu.sync_copy(xs_h, xs_v)
    pltpu.sync_copy(ys_h, ys_v)
    pltpu.sync_copy(zs_h, zs_v)
    pltpu.sync_copy(sp_h, sp_v)
    pltpu.sync_copy(t3_h, t3_v)
    pltpu.sync_copy(ta_h, ta_v)
    pltpu.sync_copy(te_h, te_v)
    pltpu.sync_copy(tf_h, tf_v)

    lanes = lax.broadcasted_iota(jnp.int32, (16,), 0)
    zero16 = jnp.zeros((16,), jnp.float32)

    chunk = wid % _SC_CHUNKS
    half = wid // _SC_CHUNKS
    if True:
        base = chunk * 16
        xi = xs_v[pl.ds(base, 16)]
        yi = ys_v[pl.ds(base, 16)]
        zi = zs_v[pl.ds(base, 16)]
        si4 = sp_v[pl.ds(base, 16)] * 4
        iv = base + lanes

        @plsc.parallel_loop(half * _SC_JSPAN, (half + 1) * _SC_JSPAN,
                            carry=(zero16, zero16, zero16, zero16),
                            unroll=4)
        def j_loop(j, carry):
            fx, fy, fz, e = carry
            jf = jnp.full((16,), j, jnp.int32)
            xj = plsc.load_gather(xs_v, [jf])
            yj = plsc.load_gather(ys_v, [jf])
            zj = plsc.load_gather(zs_v, [jf])
            sj = plsc.load_gather(sp_v, [jf])
            idx = si4 + sj
            t3g = plsc.load_gather(t3_v, [idx])
            tag = plsc.load_gather(ta_v, [idx])
            teg = plsc.load_gather(te_v, [idx])
            tfg = plsc.load_gather(tf_v, [idx])
            dx = xj - xi
            dy = yj - yi
            dz = zj - zi
            d2 = dx * dx + dy * dy + dz * dz
            y0 = plsc.bitcast(
                jnp.int32(0x5F3759DF) - (plsc.bitcast(d2, jnp.int32) >> 1),
                jnp.float32)
            hd = 0.5 * d2
            y0 = y0 * (1.5 - hd * y0 * y0)
            y0 = y0 * (1.5 - hd * y0 * y0)
            r = d2 * y0
            xr = 1.0 - r * t3g
            mask = (jf != iv) & (r < _CUTOFF) & (xr > 0.0)
            xq = jnp.where(mask, xr, 1.0)
            xb = plsc.bitcast(xq, jnp.int32)
            ef = ((xb >> 23) - 127).astype(jnp.float32)
            mb = plsc.bitcast((xb & 0x007FFFFF) | 0x3F800000, jnp.float32)
            poly = jnp.full((16,), _LOG2_COEFFS[6], jnp.float32)
            for c in _LOG2_COEFFS[5::-1]:
                poly = poly * mb + c
            lg2 = poly + ef
            p1 = jnp.exp(tag * lg2)
            e = e + jnp.where(mask, teg * p1 * xq, 0.0)
            g = jnp.where(mask, (tfg * p1) * y0, 0.0)
            return fx + g * dx, fy + g * dy, fz + g * dz, e

        fx, fy, fz, e2 = j_loop
        fxv[...] = fx
        fyv[...] = fy
        fzv[...] = fz
        ev[...] = e2

    out_base = half * _SC_ROWS + base
    pltpu.sync_copy(fxv, fx_h.at[pl.ds(out_base, 16)])
    pltpu.sync_copy(fyv, fy_h.at[pl.ds(out_base, 16)])
    pltpu.sync_copy(fzv, fz_h.at[pl.ds(out_base, 16)])
    pltpu.sync_copy(ev, e_h.at[pl.ds(wid * 16, 16)])


def _sc_call(xs, ys, zs, spec, t3_t, ta_t, te_t, tf_t):
    mesh = plsc.VectorSubcoreMesh(core_axis_name="c", subcore_axis_name="s")
    f32 = jnp.float32
    run = pl.kernel(
        _sc_body,
        mesh=mesh,
        compiler_params=pltpu.CompilerParams(needs_layout_passes=False,
                                             skip_device_barrier=True),
        out_type=[
            jax.ShapeDtypeStruct((_SC_SPLIT * _SC_ROWS,), f32),
            jax.ShapeDtypeStruct((_SC_SPLIT * _SC_ROWS,), f32),
            jax.ShapeDtypeStruct((_SC_SPLIT * _SC_ROWS,), f32),
            jax.ShapeDtypeStruct((_NW * 16,), f32),
        ],
        scratch_types=[
            pltpu.VMEM((_N,), f32),
            pltpu.VMEM((_N,), f32),
            pltpu.VMEM((_N,), f32),
            pltpu.VMEM((_N,), jnp.int32),
            pltpu.VMEM((16,), f32),
            pltpu.VMEM((16,), f32),
            pltpu.VMEM((16,), f32),
            pltpu.VMEM((16,), f32),
            pltpu.VMEM((16,), f32),
            pltpu.VMEM((16,), f32),
            pltpu.VMEM((16,), f32),
            pltpu.VMEM((16,), f32),
        ],
    )
    return run(xs, ys, zs, spec, t3_t, ta_t, te_t, tf_t)


def _tc_body(pos_ref, posT_ref, sc_ref, sr_ref, posf_ref, t3_ref, ta_ref,
             te_ref, tf_ref, e_ref, f_ref):
    pid = pl.program_id(0)
    n = posT_ref.shape[1]
    blk = pos_ref.shape[0]

    px = posT_ref[0:1, :]
    py = posT_ref[1:2, :]
    pz = posT_ref[2:3, :]
    cx = pos_ref[:, 0:1]
    cy = pos_ref[:, 1:2]
    cz = pos_ref[:, 2:3]
    dx = px - cx
    dy = py - cy
    dz = pz - cz
    d2 = dx * dx + dy * dy + dz * dz
    rinv = jax.lax.rsqrt(d2)
    r = d2 * rinv

    oh_i = (sc_ref[...] == jax.lax.broadcasted_iota(
        jnp.int32, (blk, _NSPEC), 1)).astype(jnp.float32)
    oh_j = (sr_ref[...] == jax.lax.broadcasted_iota(
        jnp.int32, (_NSPEC, n), 0)).astype(jnp.float32)

    def pair_field(t_ref):
        return jnp.dot(
            jnp.dot(oh_i, t_ref[...], preferred_element_type=jnp.float32),
            oh_j, preferred_element_type=jnp.float32)

    t3 = pair_field(t3_ref)
    ta = pair_field(ta_ref)
    te = pair_field(te_ref)
    tf = pair_field(tf_ref)

    row = _SC_ROWS + pid * blk + jax.lax.broadcasted_iota(
        jnp.int32, (blk, n), 0)
    col = jax.lax.broadcasted_iota(jnp.int32, (blk, n), 1)
    xr = 1.0 - r * t3
    mask = (row != col) & (r < _CUTOFF) & (xr > 0.0)

    xq = jnp.where(mask, xr, 1.0)
    p1 = jnp.exp(ta * jnp.log(xq))
    energies = jnp.where(mask, te * (p1 * xq), 0.0)
    g = jnp.where(mask, (tf * p1) * rinv, 0.0)

    s = jnp.sum(g, axis=1, keepdims=True)
    f_ref[...] = (jnp.dot(g, posf_ref[...], preferred_element_type=jnp.float32)
                  - s * pos_ref[...])

    @pl.when(pid == 0)
    def _():
        e_ref[...] = jnp.zeros((1, 1), jnp.float32)

    e_ref[...] += 0.5 * jnp.sum(energies, keepdims=True)


def _tc_call(positions, species):
    n = positions.shape[0]
    nrows = n - _SC_ROWS
    posT = positions.T
    spec_col = species.reshape(n, 1).astype(jnp.int32)
    spec_row = species.reshape(1, n).astype(jnp.int32)
    row0 = _SC_ROWS // _TC_BLOCK

    def run(t3_m, ta_m, te_m, tf_m):
        return pl.pallas_call(
            _tc_body,
            grid=(nrows // _TC_BLOCK,),
            compiler_params=pltpu.CompilerParams(skip_device_barrier=True),
            in_specs=[
                pl.BlockSpec((_TC_BLOCK, 3), lambda i: (row0 + i, 0)),
                pl.BlockSpec((3, n), lambda i: (0, 0)),
                pl.BlockSpec((_TC_BLOCK, 1), lambda i: (row0 + i, 0)),
                pl.BlockSpec((1, n), lambda i: (0, 0)),
                pl.BlockSpec((n, 3), lambda i: (0, 0)),
                pl.BlockSpec((_NSPEC, _NSPEC), lambda i: (0, 0)),
                pl.BlockSpec((_NSPEC, _NSPEC), lambda i: (0, 0)),
                pl.BlockSpec((_NSPEC, _NSPEC), lambda i: (0, 0)),
                pl.BlockSpec((_NSPEC, _NSPEC), lambda i: (0, 0)),
            ],
            out_specs=[
                pl.BlockSpec((1, 1), lambda i: (0, 0)),
                pl.BlockSpec((_TC_BLOCK, 3), lambda i: (i, 0)),
            ],
            out_shape=[
                jax.ShapeDtypeStruct((1, 1), jnp.float32),
                jax.ShapeDtypeStruct((nrows, 3), jnp.float32),
            ],
        )(positions, posT, spec_col, spec_row, positions,
          t3_m, ta_m, te_m, tf_m)

    return run


def kernel(positions, species, sigma_matrix, epsilon_matrix, alpha_matrix):
    xs = positions[:, 0]
    ys = positions[:, 1]
    zs = positions[:, 2]
    spec = species.astype(jnp.int32)
    t3_t = (1.0 / sigma_matrix).reshape(-1)
    ta_t = ((alpha_matrix - 1.0) * _LN2).reshape(-1)
    te_t = (epsilon_matrix / alpha_matrix).reshape(-1)
    tf_t = (-epsilon_matrix / sigma_matrix).reshape(-1)
    fxp, fyp, fzp, e_part = _sc_call(xs, ys, zs, spec, t3_t, ta_t, te_t, tf_t)
    fx = jnp.sum(fxp.reshape(_SC_SPLIT, _SC_ROWS), axis=0)
    fy = jnp.sum(fyp.reshape(_SC_SPLIT, _SC_ROWS), axis=0)
    fz = jnp.sum(fzp.reshape(_SC_SPLIT, _SC_ROWS), axis=0)
    e_tc, f_tc = _tc_call(positions, species)(
        1.0 / sigma_matrix, alpha_matrix - 1.0,
        epsilon_matrix / alpha_matrix, -epsilon_matrix / sigma_matrix)
    forces = jnp.concatenate(
        [jnp.stack([fx, fy, fz], axis=1), f_tc], axis=0)
    energy = 0.5 * jnp.sum(e_part) + e_tc[0, 0]
    return energy, forces


# speedup vs baseline: 1.0133x; 1.0075x over previous
import jax
import jax.numpy as jnp
from jax import lax
from jax.experimental import pallas as pl
from jax.experimental.pallas import tpu as pltpu
from jax.experimental.pallas import tpu_sc as plsc

_N = 2048
_NSPEC = 4
_CUTOFF = 1.0
_NW = 32
_LN2 = 0.6931471805599453

_SC_ROWS = 256
_SC_CHUNKS = _SC_ROWS // 16
_SC_SPLIT = _NW // _SC_CHUNKS
_SC_JSPAN = _N // _SC_SPLIT
_TC_BLOCK = 256

_LOG2_COEFFS = (
    -3.0283174810372375, 6.065830143177264, -5.2641104770701075,
    3.218832837050299, -1.2342631730323361, 0.26685882285942003,
    -0.024825606614202734,
)


def _sc_body(xs_h, ys_h, zs_h, sp_h, t3_h, ta_h, te_h, tf_h,
             fx_h, fy_h, fz_h, e_h,
             xs_v, ys_v, zs_v, sp_v, t3_v, ta_v, te_v, tf_v,
             fxv, fyv, fzv, ev):
    wid = lax.axis_index("s") * 2 + lax.axis_index("c")
    pltpu.sync_copy(xs_h, xs_v)
    pltpu.sync_copy(ys_h, ys_v)
    pltpu.sync_copy(zs_h, zs_v)
    pltpu.sync_copy(sp_h, sp_v)
    pltpu.sync_copy(t3_h, t3_v)
    pltpu.sync_copy(ta_h, ta_v)
    pltpu.sync_copy(te_h, te_v)
    pltpu.sync_copy(tf_h, tf_v)

    lanes = lax.broadcasted_iota(jnp.int32, (16,), 0)
    zero16 = jnp.zeros((16,), jnp.float32)

    chunk = wid % _SC_CHUNKS
    half = wid // _SC_CHUNKS
    if True:
        base = chunk * 16
        xi = xs_v[pl.ds(base, 16)]
        yi = ys_v[pl.ds(base, 16)]
        zi = zs_v[pl.ds(base, 16)]
        si4 = sp_v[pl.ds(base, 16)] * 4
        iv = base + lanes

        @plsc.parallel_loop(half * _SC_JSPAN, (half + 1) * _SC_JSPAN,
                            carry=(zero16, zero16, zero16, zero16),
                            unroll=4)
        def j_loop(j, carry):
            fx, fy, fz, e = carry
            jf = jnp.full((16,), j, jnp.int32)
            xj = plsc.load_gather(xs_v, [jf])
            yj = plsc.load_gather(ys_v, [jf])
            zj = plsc.load_gather(zs_v, [jf])
            sj = plsc.load_gather(sp_v, [jf])
            idx = si4 + sj
            t3g = plsc.load_gather(t3_v, [idx])
            tag = plsc.load_gather(ta_v, [idx])
            teg = plsc.load_gather(te_v, [idx])
            tfg = plsc.load_gather(tf_v, [idx])
            dx = xj - xi
            dy = yj - yi
            dz = zj - zi
            d2 = dx * dx + dy * dy + dz * dz
            y0 = plsc.bitcast(
                jnp.int32(0x5F3759DF) - (plsc.bitcast(d2, jnp.int32) >> 1),
                jnp.float32)
            hd = 0.5 * d2
            y0 = y0 * (1.5 - hd * y0 * y0)
            y0 = y0 * (1.5 - hd * y0 * y0)
            r = d2 * y0
            xr = 1.0 - r * t3g
            mask = (jf != iv) & (r < _CUTOFF) & (xr > 0.0)
            xq = jnp.where(mask, xr, 1.0)
            xb = plsc.bitcast(xq, jnp.int32)
            ef = ((xb >> 23) - 127).astype(jnp.float32)
            mb = plsc.bitcast((xb & 0x007FFFFF) | 0x3F800000, jnp.float32)
            poly = jnp.full((16,), _LOG2_COEFFS[6], jnp.float32)
            for c in _LOG2_COEFFS[5::-1]:
                poly = poly * mb + c
            lg2 = poly + ef
            p1 = jnp.exp(tag * lg2)
            e = e + jnp.where(mask, teg * p1 * xq, 0.0)
            g = jnp.where(mask, (tfg * p1) * y0, 0.0)
            return fx + g * dx, fy + g * dy, fz + g * dz, e

        fx, fy, fz, e2 = j_loop
        fxv[...] = fx
        fyv[...] = fy
        fzv[...] = fz
        ev[...] = e2

    out_base = half * _SC_ROWS + base
    pltpu.sync_copy(fxv, fx_h.at[pl.ds(out_base, 16)])
    pltpu.sync_copy(fyv, fy_h.at[pl.ds(out_base, 16)])
    pltpu.sync_copy(fzv, fz_h.at[pl.ds(out_base, 16)])
    pltpu.sync_copy(ev, e_h.at[pl.ds(wid * 16, 16)])


def _sc_call(xs, ys, zs, spec, t3_t, ta_t, te_t, tf_t):
    mesh = plsc.VectorSubcoreMesh(core_axis_name="c", subcore_axis_name="s")
    f32 = jnp.float32
    run = pl.kernel(
        _sc_body,
        mesh=mesh,
        compiler_params=pltpu.CompilerParams(needs_layout_passes=False,
                                             skip_device_barrier=True),
        out_type=[
            jax.ShapeDtypeStruct((_SC_SPLIT * _SC_ROWS,), f32),
            jax.ShapeDtypeStruct((_SC_SPLIT * _SC_ROWS,), f32),
            jax.ShapeDtypeStruct((_SC_SPLIT * _SC_ROWS,), f32),
            jax.ShapeDtypeStruct((_NW * 16,), f32),
        ],
        scratch_types=[
            pltpu.VMEM((_N,), f32),
            pltpu.VMEM((_N,), f32),
            pltpu.VMEM((_N,), f32),
            pltpu.VMEM((_N,), jnp.int32),
            pltpu.VMEM((16,), f32),
            pltpu.VMEM((16,), f32),
            pltpu.VMEM((16,), f32),
            pltpu.VMEM((16,), f32),
            pltpu.VMEM((16,), f32),
            pltpu.VMEM((16,), f32),
            pltpu.VMEM((16,), f32),
            pltpu.VMEM((16,), f32),
        ],
    )
    return run(xs, ys, zs, spec, t3_t, ta_t, te_t, tf_t)


def _tc_body(pos_ref, posT_ref, sc_ref, sr_ref, posf_ref, sig_ref, alp_ref,
             eps_ref, e_ref, f_ref):
    pid = pl.program_id(0)
    n = posT_ref.shape[1]
    blk = pos_ref.shape[0]

    px = posT_ref[0:1, :]
    py = posT_ref[1:2, :]
    pz = posT_ref[2:3, :]
    cx = pos_ref[:, 0:1]
    cy = pos_ref[:, 1:2]
    cz = pos_ref[:, 2:3]
    dx = px - cx
    dy = py - cy
    dz = pz - cz
    d2 = dx * dx + dy * dy + dz * dz
    rinv = jax.lax.rsqrt(d2)
    r = d2 * rinv

    oh_i = (sc_ref[...] == jax.lax.broadcasted_iota(
        jnp.int32, (blk, _NSPEC), 1)).astype(jnp.float32)
    oh_j = (sr_ref[...] == jax.lax.broadcasted_iota(
        jnp.int32, (_NSPEC, n), 0)).astype(jnp.float32)

    def pair_field(t44):
        return jnp.dot(
            jnp.dot(oh_i, t44, preferred_element_type=jnp.float32),
            oh_j, preferred_element_type=jnp.float32)

    t3 = pair_field(1.0 / sig_ref[...])
    ta = pair_field(alp_ref[...] - 1.0)
    te = pair_field(eps_ref[...] / alp_ref[...])
    tf = pair_field(-eps_ref[...] / sig_ref[...])

    xr = 1.0 - r * t3
    mask = (d2 > 0.0) & (r < _CUTOFF) & (xr > 0.0)

    xq = jnp.where(mask, xr, 1.0)
    p1 = jnp.exp(ta * jnp.log(xq))
    energies = jnp.where(mask, te * (p1 * xq), 0.0)
    g = jnp.where(mask, (tf * p1) * rinv, 0.0)

    s = jnp.sum(g, axis=1, keepdims=True)
    f_ref[...] = (jnp.dot(g, posf_ref[...], preferred_element_type=jnp.float32)
                  - s * pos_ref[...])

    @pl.when(pid == 0)
    def _():
        e_ref[...] = jnp.zeros((1, 1), jnp.float32)

    e_ref[...] += 0.5 * jnp.sum(energies, keepdims=True)


def _tc_call(positions, species):
    n = positions.shape[0]
    nrows = n - _SC_ROWS
    posT = positions.T
    spec_col = species.reshape(n, 1).astype(jnp.int32)
    spec_row = species.reshape(1, n).astype(jnp.int32)
    row0 = _SC_ROWS // _TC_BLOCK

    def run(sig_m, alp_m, eps_m):
        return pl.pallas_call(
            _tc_body,
            grid=(nrows // _TC_BLOCK,),
            compiler_params=pltpu.CompilerParams(skip_device_barrier=True),
            in_specs=[
                pl.BlockSpec((_TC_BLOCK, 3), lambda i: (row0 + i, 0)),
                pl.BlockSpec((3, n), lambda i: (0, 0)),
                pl.BlockSpec((_TC_BLOCK, 1), lambda i: (row0 + i, 0)),
                pl.BlockSpec((1, n), lambda i: (0, 0)),
                pl.BlockSpec((n, 3), lambda i: (0, 0)),
                pl.BlockSpec((_NSPEC, _NSPEC), lambda i: (0, 0)),
                pl.BlockSpec((_NSPEC, _NSPEC), lambda i: (0, 0)),
                pl.BlockSpec((_NSPEC, _NSPEC), lambda i: (0, 0)),
            ],
            out_specs=[
                pl.BlockSpec((1, 1), lambda i: (0, 0)),
                pl.BlockSpec((_TC_BLOCK, 3), lambda i: (i, 0)),
            ],
            out_shape=[
                jax.ShapeDtypeStruct((1, 1), jnp.float32),
                jax.ShapeDtypeStruct((nrows, 3), jnp.float32),
            ],
        )(positions, posT, spec_col, spec_row, positions,
          sig_m, alp_m, eps_m)

    return run


def kernel(positions, species, sigma_matrix, epsilon_matrix, alpha_matrix):
    xs = positions[:, 0]
    ys = positions[:, 1]
    zs = positions[:, 2]
    spec = species.astype(jnp.int32)
    t3_t = (1.0 / sigma_matrix).reshape(-1)
    ta_t = ((alpha_matrix - 1.0) * _LN2).reshape(-1)
    te_t = (epsilon_matrix / alpha_matrix).reshape(-1)
    tf_t = (-epsilon_matrix / sigma_matrix).reshape(-1)
    fxp, fyp, fzp, e_part = _sc_call(xs, ys, zs, spec, t3_t, ta_t, te_t, tf_t)
    fx = jnp.sum(fxp.reshape(_SC_SPLIT, _SC_ROWS), axis=0)
    fy = jnp.sum(fyp.reshape(_SC_SPLIT, _SC_ROWS), axis=0)
    fz = jnp.sum(fzp.reshape(_SC_SPLIT, _SC_ROWS), axis=0)
    e_tc, f_tc = _tc_call(positions, species)(
        sigma_matrix, alpha_matrix, epsilon_matrix)
    forces = jnp.concatenate(
        [jnp.stack([fx, fy, fz], axis=1), f_tc], axis=0)
    energy = 0.5 * jnp.sum(e_part) + e_tc[0, 0]
    return energy, forces


# speedup vs baseline: 1.0449x; 1.0311x over previous
import jax
import jax.numpy as jnp
from jax import lax
from jax.experimental import pallas as pl
from jax.experimental.pallas import tpu as pltpu
from jax.experimental.pallas import tpu_sc as plsc

_N = 2048
_NSPEC = 4
_CUTOFF = 1.0
_NW = 32
_LN2 = 0.6931471805599453

_SC_ROWS = 256
_SC_CHUNKS = _SC_ROWS // 16
_SC_SPLIT = _NW // _SC_CHUNKS
_SC_JSPAN = _N // _SC_SPLIT
_TC_BLOCK = 448
_TC_ROW0 = 0

_LOG2_COEFFS = (
    -3.0283174810372375, 6.065830143177264, -5.2641104770701075,
    3.218832837050299, -1.2342631730323361, 0.26685882285942003,
    -0.024825606614202734,
)


def _sc_body(xs_h, ys_h, zs_h, sp_h, t3_h, ta_h, te_h, tf_h,
             fx_h, fy_h, fz_h, e_h,
             xs_v, ys_v, zs_v, sp_v, t3_v, ta_v, te_v, tf_v,
             fxv, fyv, fzv, ev):
    wid = lax.axis_index("s") * 2 + lax.axis_index("c")
    pltpu.sync_copy(xs_h, xs_v)
    pltpu.sync_copy(ys_h, ys_v)
    pltpu.sync_copy(zs_h, zs_v)
    pltpu.sync_copy(sp_h, sp_v)
    pltpu.sync_copy(t3_h, t3_v)
    pltpu.sync_copy(ta_h, ta_v)
    pltpu.sync_copy(te_h, te_v)
    pltpu.sync_copy(tf_h, tf_v)

    lanes = lax.broadcasted_iota(jnp.int32, (16,), 0)
    zero16 = jnp.zeros((16,), jnp.float32)

    chunk = wid % _SC_CHUNKS
    half = wid // _SC_CHUNKS
    if True:
        base = (_N - _SC_ROWS) + chunk * 16
        xi = xs_v[pl.ds(base, 16)]
        yi = ys_v[pl.ds(base, 16)]
        zi = zs_v[pl.ds(base, 16)]
        si4 = sp_v[pl.ds(base, 16)] * 4
        iv = base + lanes

        @plsc.parallel_loop(half * _SC_JSPAN, (half + 1) * _SC_JSPAN,
                            carry=(zero16, zero16, zero16, zero16),
                            unroll=4)
        def j_loop(j, carry):
            fx, fy, fz, e = carry
            jf = jnp.full((16,), j, jnp.int32)
            xj = plsc.load_gather(xs_v, [jf])
            yj = plsc.load_gather(ys_v, [jf])
            zj = plsc.load_gather(zs_v, [jf])
            sj = plsc.load_gather(sp_v, [jf])
            idx = si4 + sj
            t3g = plsc.load_gather(t3_v, [idx])
            tag = plsc.load_gather(ta_v, [idx])
            teg = plsc.load_gather(te_v, [idx])
            tfg = plsc.load_gather(tf_v, [idx])
            dx = xj - xi
            dy = yj - yi
            dz = zj - zi
            d2 = dx * dx + dy * dy + dz * dz
            y0 = plsc.bitcast(
                jnp.int32(0x5F3759DF) - (plsc.bitcast(d2, jnp.int32) >> 1),
                jnp.float32)
            hd = 0.5 * d2
            y0 = y0 * (1.5 - hd * y0 * y0)
            y0 = y0 * (1.5 - hd * y0 * y0)
            r = d2 * y0
            xr = 1.0 - r * t3g
            mask = (jf != iv) & (r < _CUTOFF) & (xr > 0.0)
            xq = jnp.where(mask, xr, 1.0)
            xb = plsc.bitcast(xq, jnp.int32)
            ef = ((xb >> 23) - 127).astype(jnp.float32)
            mb = plsc.bitcast((xb & 0x007FFFFF) | 0x3F800000, jnp.float32)
            poly = jnp.full((16,), _LOG2_COEFFS[6], jnp.float32)
            for c in _LOG2_COEFFS[5::-1]:
                poly = poly * mb + c
            lg2 = poly + ef
            p1 = jnp.exp(tag * lg2)
            e = e + jnp.where(mask, teg * p1 * xq, 0.0)
            g = jnp.where(mask, (tfg * p1) * y0, 0.0)
            return fx + g * dx, fy + g * dy, fz + g * dz, e

        fx, fy, fz, e2 = j_loop
        fxv[...] = fx
        fyv[...] = fy
        fzv[...] = fz
        ev[...] = e2

    out_base = half * _SC_ROWS + (base - (_N - _SC_ROWS))
    pltpu.sync_copy(fxv, fx_h.at[pl.ds(out_base, 16)])
    pltpu.sync_copy(fyv, fy_h.at[pl.ds(out_base, 16)])
    pltpu.sync_copy(fzv, fz_h.at[pl.ds(out_base, 16)])
    pltpu.sync_copy(ev, e_h.at[pl.ds(wid * 16, 16)])


def _sc_call(xs, ys, zs, spec, t3_t, ta_t, te_t, tf_t):
    mesh = plsc.VectorSubcoreMesh(core_axis_name="c", subcore_axis_name="s")
    f32 = jnp.float32
    run = pl.kernel(
        _sc_body,
        mesh=mesh,
        compiler_params=pltpu.CompilerParams(needs_layout_passes=False,
                                             skip_device_barrier=True),
        out_type=[
            jax.ShapeDtypeStruct((_SC_SPLIT * _SC_ROWS,), f32),
            jax.ShapeDtypeStruct((_SC_SPLIT * _SC_ROWS,), f32),
            jax.ShapeDtypeStruct((_SC_SPLIT * _SC_ROWS,), f32),
            jax.ShapeDtypeStruct((_NW * 16,), f32),
        ],
        scratch_types=[
            pltpu.VMEM((_N,), f32),
            pltpu.VMEM((_N,), f32),
            pltpu.VMEM((_N,), f32),
            pltpu.VMEM((_N,), jnp.int32),
            pltpu.VMEM((16,), f32),
            pltpu.VMEM((16,), f32),
            pltpu.VMEM((16,), f32),
            pltpu.VMEM((16,), f32),
            pltpu.VMEM((16,), f32),
            pltpu.VMEM((16,), f32),
            pltpu.VMEM((16,), f32),
            pltpu.VMEM((16,), f32),
        ],
    )
    return run(xs, ys, zs, spec, t3_t, ta_t, te_t, tf_t)


def _tc_body(pos_ref, posT_ref, sc_ref, sr_ref, posf_ref, sig_ref, alp_ref,
             eps_ref, e_ref, f_ref):
    pid = pl.program_id(0)
    n = posT_ref.shape[1]
    blk = pos_ref.shape[0]

    px = posT_ref[0:1, :]
    py = posT_ref[1:2, :]
    pz = posT_ref[2:3, :]
    cx = pos_ref[:, 0:1]
    cy = pos_ref[:, 1:2]
    cz = pos_ref[:, 2:3]
    dx = px - cx
    dy = py - cy
    dz = pz - cz
    d2 = dx * dx + dy * dy + dz * dz
    rinv = jax.lax.rsqrt(d2)
    r = d2 * rinv

    oh_i = (sc_ref[...] == jax.lax.broadcasted_iota(
        jnp.int32, (blk, _NSPEC), 1)).astype(jnp.float32)
    oh_j = (sr_ref[...] == jax.lax.broadcasted_iota(
        jnp.int32, (_NSPEC, n), 0)).astype(jnp.float32)

    def pair_field(t44):
        return jnp.dot(
            jnp.dot(oh_i, t44, preferred_element_type=jnp.float32),
            oh_j, preferred_element_type=jnp.float32)

    t3 = pair_field(1.0 / sig_ref[...])
    ta = pair_field(alp_ref[...] - 1.0)
    te = pair_field(eps_ref[...] / alp_ref[...])
    tf = pair_field(-eps_ref[...] / sig_ref[...])

    xr = 1.0 - r * t3
    mask = (d2 > 0.0) & (r < _CUTOFF) & (xr > 0.0)

    xq = jnp.where(mask, xr, 1.0)
    p1 = jnp.exp(ta * jnp.log(xq))
    energies = jnp.where(mask, te * (p1 * xq), 0.0)
    g = jnp.where(mask, (tf * p1) * rinv, 0.0)

    s = jnp.sum(g, axis=1, keepdims=True)
    f_ref[...] = (jnp.dot(g, posf_ref[...], preferred_element_type=jnp.float32)
                  - s * pos_ref[...])

    @pl.when(pid == 0)
    def _():
        e_ref[...] = jnp.zeros((1, 1), jnp.float32)

    e_ref[...] += 0.5 * jnp.sum(energies, keepdims=True)


def _tc_call(positions, species):
    n = positions.shape[0]
    nrows = n - _SC_ROWS
    posT = positions.T
    spec_col = species.reshape(n, 1).astype(jnp.int32)
    spec_row = species.reshape(1, n).astype(jnp.int32)

    def run(sig_m, alp_m, eps_m):
        return pl.pallas_call(
            _tc_body,
            grid=(nrows // _TC_BLOCK,),
            compiler_params=pltpu.CompilerParams(skip_device_barrier=True),
            in_specs=[
                pl.BlockSpec((_TC_BLOCK, 3), lambda i: (i, 0)),
                pl.BlockSpec((3, n), lambda i: (0, 0)),
                pl.BlockSpec((_TC_BLOCK, 1), lambda i: (i, 0)),
                pl.BlockSpec((1, n), lambda i: (0, 0)),
                pl.BlockSpec((n, 3), lambda i: (0, 0)),
                pl.BlockSpec((_NSPEC, _NSPEC), lambda i: (0, 0)),
                pl.BlockSpec((_NSPEC, _NSPEC), lambda i: (0, 0)),
                pl.BlockSpec((_NSPEC, _NSPEC), lambda i: (0, 0)),
            ],
            out_specs=[
                pl.BlockSpec((1, 1), lambda i: (0, 0)),
                pl.BlockSpec((_TC_BLOCK, 3), lambda i: (i, 0)),
            ],
            out_shape=[
                jax.ShapeDtypeStruct((1, 1), jnp.float32),
                jax.ShapeDtypeStruct((nrows, 3), jnp.float32),
            ],
        )(positions, posT, spec_col, spec_row, positions,
          sig_m, alp_m, eps_m)

    return run


def kernel(positions, species, sigma_matrix, epsilon_matrix, alpha_matrix):
    xs = positions[:, 0]
    ys = positions[:, 1]
    zs = positions[:, 2]
    spec = species.astype(jnp.int32)
    t3_t = (1.0 / sigma_matrix).reshape(-1)
    ta_t = ((alpha_matrix - 1.0) * _LN2).reshape(-1)
    te_t = (epsilon_matrix / alpha_matrix).reshape(-1)
    tf_t = (-epsilon_matrix / sigma_matrix).reshape(-1)
    fxp, fyp, fzp, e_part = _sc_call(xs, ys, zs, spec, t3_t, ta_t, te_t, tf_t)
    fx = jnp.sum(fxp.reshape(_SC_SPLIT, _SC_ROWS), axis=0)
    fy = jnp.sum(fyp.reshape(_SC_SPLIT, _SC_ROWS), axis=0)
    fz = jnp.sum(fzp.reshape(_SC_SPLIT, _SC_ROWS), axis=0)
    e_tc, f_tc = _tc_call(positions, species)(
        sigma_matrix, alpha_matrix, epsilon_matrix)
    forces = jnp.concatenate(
        [f_tc, jnp.stack([fx, fy, fz], axis=1)], axis=0)
    energy = 0.5 * jnp.sum(e_part) + e_tc[0, 0]
    return energy, forces


# speedup vs baseline: 1.0731x; 1.0270x over previous
import jax
import jax.numpy as jnp
from jax import lax
from jax.experimental import pallas as pl
from jax.experimental.pallas import tpu as pltpu
from jax.experimental.pallas import tpu_sc as plsc

_N = 2048
_NSPEC = 4
_CUTOFF = 1.0
_NW = 32
_LN2 = 0.6931471805599453

_SC_ROWS = 256
_SC_CHUNKS = _SC_ROWS // 16
_SC_SPLIT = _NW // _SC_CHUNKS
_SC_JSPAN = _N // _SC_SPLIT
_TC_BLOCK = 896
_TC_ROW0 = 0

_LOG2_COEFFS = (
    -3.0283174810372375, 6.065830143177264, -5.2641104770701075,
    3.218832837050299, -1.2342631730323361, 0.26685882285942003,
    -0.024825606614202734,
)


def _sc_body(xs_h, ys_h, zs_h, sp_h, t3_h, ta_h, te_h, tf_h,
             fx_h, fy_h, fz_h, e_h,
             xs_v, ys_v, zs_v, sp_v, t3_v, ta_v, te_v, tf_v,
             fxv, fyv, fzv, ev):
    wid = lax.axis_index("s") * 2 + lax.axis_index("c")
    pltpu.sync_copy(xs_h, xs_v)
    pltpu.sync_copy(ys_h, ys_v)
    pltpu.sync_copy(zs_h, zs_v)
    pltpu.sync_copy(sp_h, sp_v)
    pltpu.sync_copy(t3_h, t3_v)
    pltpu.sync_copy(ta_h, ta_v)
    pltpu.sync_copy(te_h, te_v)
    pltpu.sync_copy(tf_h, tf_v)

    lanes = lax.broadcasted_iota(jnp.int32, (16,), 0)
    zero16 = jnp.zeros((16,), jnp.float32)

    chunk = wid % _SC_CHUNKS
    half = wid // _SC_CHUNKS
    if True:
        base = (_N - _SC_ROWS) + chunk * 16
        xi = xs_v[pl.ds(base, 16)]
        yi = ys_v[pl.ds(base, 16)]
        zi = zs_v[pl.ds(base, 16)]
        si4 = sp_v[pl.ds(base, 16)] * 4
        iv = base + lanes

        @plsc.parallel_loop(half * _SC_JSPAN, (half + 1) * _SC_JSPAN,
                            carry=(zero16, zero16, zero16, zero16),
                            unroll=4)
        def j_loop(j, carry):
            fx, fy, fz, e = carry
            jf = jnp.full((16,), j, jnp.int32)
            xj = plsc.load_gather(xs_v, [jf])
            yj = plsc.load_gather(ys_v, [jf])
            zj = plsc.load_gather(zs_v, [jf])
            sj = plsc.load_gather(sp_v, [jf])
            idx = si4 + sj
            t3g = plsc.load_gather(t3_v, [idx])
            tag = plsc.load_gather(ta_v, [idx])
            teg = plsc.load_gather(te_v, [idx])
            tfg = plsc.load_gather(tf_v, [idx])
            dx = xj - xi
            dy = yj - yi
            dz = zj - zi
            d2 = dx * dx + dy * dy + dz * dz
            y0 = plsc.bitcast(
                jnp.int32(0x5F3759DF) - (plsc.bitcast(d2, jnp.int32) >> 1),
                jnp.float32)
            hd = 0.5 * d2
            y0 = y0 * (1.5 - hd * y0 * y0)
            y0 = y0 * (1.5 - hd * y0 * y0)
            r = d2 * y0
            xr = 1.0 - r * t3g
            mask = (jf != iv) & (r < _CUTOFF) & (xr > 0.0)
            xq = jnp.where(mask, xr, 1.0)
            xb = plsc.bitcast(xq, jnp.int32)
            ef = ((xb >> 23) - 127).astype(jnp.float32)
            mb = plsc.bitcast((xb & 0x007FFFFF) | 0x3F800000, jnp.float32)
            poly = jnp.full((16,), _LOG2_COEFFS[6], jnp.float32)
            for c in _LOG2_COEFFS[5::-1]:
                poly = poly * mb + c
            lg2 = poly + ef
            p1 = jnp.exp(tag * lg2)
            e = e + jnp.where(mask, teg * p1 * xq, 0.0)
            g = jnp.where(mask, (tfg * p1) * y0, 0.0)
            return fx + g * dx, fy + g * dy, fz + g * dz, e

        fx, fy, fz, e2 = j_loop
        fxv[...] = fx
        fyv[...] = fy
        fzv[...] = fz
        ev[...] = e2

    out_base = half * _SC_ROWS + (base - (_N - _SC_ROWS))
    pltpu.sync_copy(fxv, fx_h.at[pl.ds(out_base, 16)])
    pltpu.sync_copy(fyv, fy_h.at[pl.ds(out_base, 16)])
    pltpu.sync_copy(fzv, fz_h.at[pl.ds(out_base, 16)])
    pltpu.sync_copy(ev, e_h.at[pl.ds(wid * 16, 16)])


def _sc_call(xs, ys, zs, spec, t3_t, ta_t, te_t, tf_t):
    mesh = plsc.VectorSubcoreMesh(core_axis_name="c", subcore_axis_name="s")
    f32 = jnp.float32
    run = pl.kernel(
        _sc_body,
        mesh=mesh,
        compiler_params=pltpu.CompilerParams(needs_layout_passes=False,
                                             skip_device_barrier=True),
        out_type=[
            jax.ShapeDtypeStruct((_SC_SPLIT * _SC_ROWS,), f32),
            jax.ShapeDtypeStruct((_SC_SPLIT * _SC_ROWS,), f32),
            jax.ShapeDtypeStruct((_SC_SPLIT * _SC_ROWS,), f32),
            jax.ShapeDtypeStruct((_NW * 16,), f32),
        ],
        scratch_types=[
            pltpu.VMEM((_N,), f32),
            pltpu.VMEM((_N,), f32),
            pltpu.VMEM((_N,), f32),
            pltpu.VMEM((_N,), jnp.int32),
            pltpu.VMEM((16,), f32),
            pltpu.VMEM((16,), f32),
            pltpu.VMEM((16,), f32),
            pltpu.VMEM((16,), f32),
            pltpu.VMEM((16,), f32),
            pltpu.VMEM((16,), f32),
            pltpu.VMEM((16,), f32),
            pltpu.VMEM((16,), f32),
        ],
    )
    return run(xs, ys, zs, spec, t3_t, ta_t, te_t, tf_t)


def _tc_body(pos_ref, posT_ref, sc_ref, sr_ref, posf_ref, sig_ref, alp_ref,
             eps_ref, e_ref, f_ref):
    pid = pl.program_id(0)
    n = posT_ref.shape[1]
    blk = pos_ref.shape[0]

    px = posT_ref[0:1, :]
    py = posT_ref[1:2, :]
    pz = posT_ref[2:3, :]
    cx = pos_ref[:, 0:1]
    cy = pos_ref[:, 1:2]
    cz = pos_ref[:, 2:3]
    dx = px - cx
    dy = py - cy
    dz = pz - cz
    d2 = dx * dx + dy * dy + dz * dz
    rinv = jax.lax.rsqrt(d2)
    r = d2 * rinv

    oh_i = (sc_ref[...] == jax.lax.broadcasted_iota(
        jnp.int32, (blk, _NSPEC), 1)).astype(jnp.float32)
    oh_j = (sr_ref[...] == jax.lax.broadcasted_iota(
        jnp.int32, (_NSPEC, n), 0)).astype(jnp.float32)

    def pair_field(t44):
        return jnp.dot(
            jnp.dot(oh_i, t44, preferred_element_type=jnp.float32),
            oh_j, preferred_element_type=jnp.float32)

    t3 = pair_field(1.0 / sig_ref[...])
    ta = pair_field(alp_ref[...] - 1.0)
    te = pair_field(eps_ref[...] / alp_ref[...])
    tf = pair_field(-eps_ref[...] / sig_ref[...])

    xr = 1.0 - r * t3
    mask = (d2 > 0.0) & (r < _CUTOFF) & (xr > 0.0)

    xq = jnp.where(mask, xr, 1.0)
    p1 = jnp.exp(ta * jnp.log(xq))
    energies = jnp.where(mask, te * (p1 * xq), 0.0)
    g = jnp.where(mask, (tf * p1) * rinv, 0.0)

    s = jnp.sum(g, axis=1, keepdims=True)
    f_ref[...] = (jnp.dot(g, posf_ref[...], preferred_element_type=jnp.float32)
                  - s * pos_ref[...])

    @pl.when(pid == 0)
    def _():
        e_ref[...] = jnp.zeros((1, 1), jnp.float32)

    e_ref[...] += 0.5 * jnp.sum(energies, keepdims=True)


def _tc_call(positions, species):
    n = positions.shape[0]
    nrows = n - _SC_ROWS
    posT = positions.T
    spec_col = species.reshape(n, 1).astype(jnp.int32)
    spec_row = species.reshape(1, n).astype(jnp.int32)

    def run(sig_m, alp_m, eps_m):
        return pl.pallas_call(
            _tc_body,
            grid=(nrows // _TC_BLOCK,),
            compiler_params=pltpu.CompilerParams(skip_device_barrier=True),
            in_specs=[
                pl.BlockSpec((_TC_BLOCK, 3), lambda i: (i, 0)),
                pl.BlockSpec((3, n), lambda i: (0, 0)),
                pl.BlockSpec((_TC_BLOCK, 1), lambda i: (i, 0)),
                pl.BlockSpec((1, n), lambda i: (0, 0)),
                pl.BlockSpec((n, 3), lambda i: (0, 0)),
                pl.BlockSpec((_NSPEC, _NSPEC), lambda i: (0, 0)),
                pl.BlockSpec((_NSPEC, _NSPEC), lambda i: (0, 0)),
                pl.BlockSpec((_NSPEC, _NSPEC), lambda i: (0, 0)),
            ],
            out_specs=[
                pl.BlockSpec((1, 1), lambda i: (0, 0)),
                pl.BlockSpec((_TC_BLOCK, 3), lambda i: (i, 0)),
            ],
            out_shape=[
                jax.ShapeDtypeStruct((1, 1), jnp.float32),
                jax.ShapeDtypeStruct((nrows, 3), jnp.float32),
            ],
        )(positions, posT, spec_col, spec_row, positions,
          sig_m, alp_m, eps_m)

    return run


def kernel(positions, species, sigma_matrix, epsilon_matrix, alpha_matrix):
    xs = positions[:, 0]
    ys = positions[:, 1]
    zs = positions[:, 2]
    spec = species.astype(jnp.int32)
    t3_t = (1.0 / sigma_matrix).reshape(-1)
    ta_t = ((alpha_matrix - 1.0) * _LN2).reshape(-1)
    te_t = (epsilon_matrix / alpha_matrix).reshape(-1)
    tf_t = (-epsilon_matrix / sigma_matrix).reshape(-1)
    fxp, fyp, fzp, e_part = _sc_call(xs, ys, zs, spec, t3_t, ta_t, te_t, tf_t)
    fx = jnp.sum(fxp.reshape(_SC_SPLIT, _SC_ROWS), axis=0)
    fy = jnp.sum(fyp.reshape(_SC_SPLIT, _SC_ROWS), axis=0)
    fz = jnp.sum(fzp.reshape(_SC_SPLIT, _SC_ROWS), axis=0)
    e_tc, f_tc = _tc_call(positions, species)(
        sigma_matrix, alpha_matrix, epsilon_matrix)
    forces = jnp.concatenate(
        [f_tc, jnp.stack([fx, fy, fz], axis=1)], axis=0)
    energy = 0.5 * jnp.sum(e_part) + e_tc[0, 0]
    return energy, forces


# speedup vs baseline: 1.0746x; 1.0014x over previous
import jax
import jax.numpy as jnp
from jax import lax
from jax.experimental import pallas as pl
from jax.experimental.pallas import tpu as pltpu
from jax.experimental.pallas import tpu_sc as plsc

_N = 2048
_NSPEC = 4
_CUTOFF = 1.0
_NW = 32
_LN2 = 0.6931471805599453

_SC_ROWS = 256
_SC_CHUNKS = _SC_ROWS // 16
_SC_SPLIT = _NW // _SC_CHUNKS
_SC_JSPAN = _N // _SC_SPLIT
_TC_BLOCK = 896

_LOG2_COEFFS = (
    -3.0283174810372375, 6.065830143177264, -5.2641104770701075,
    3.218832837050299, -1.2342631730323361, 0.26685882285942003,
    -0.024825606614202734,
)


def _sc_body(xs_h, ys_h, zs_h, sp_h, t3_h, ta_h, te_h, tf_h,
             fx_h, fy_h, fz_h, e_h,
             xs_v, ys_v, zs_v, sp_v, t3_v, ta_v, te_v, tf_v,
             fxv, fyv, fzv, ev):
    wid = lax.axis_index("s") * 2 + lax.axis_index("c")
    pltpu.sync_copy(xs_h, xs_v)
    pltpu.sync_copy(ys_h, ys_v)
    pltpu.sync_copy(zs_h, zs_v)
    pltpu.sync_copy(sp_h, sp_v)
    pltpu.sync_copy(t3_h, t3_v)
    pltpu.sync_copy(ta_h, ta_v)
    pltpu.sync_copy(te_h, te_v)
    pltpu.sync_copy(tf_h, tf_v)

    lanes = lax.broadcasted_iota(jnp.int32, (16,), 0)
    zero16 = jnp.zeros((16,), jnp.float32)

    chunk = wid % _SC_CHUNKS
    half = wid // _SC_CHUNKS
    base = (_N - _SC_ROWS) + chunk * 16
    xi = xs_v[pl.ds(base, 16)]
    yi = ys_v[pl.ds(base, 16)]
    zi = zs_v[pl.ds(base, 16)]
    si4 = sp_v[pl.ds(base, 16)] * 4
    iv = base + lanes

    @plsc.parallel_loop(half * _SC_JSPAN, (half + 1) * _SC_JSPAN,
                        carry=(zero16, zero16, zero16, zero16),
                        unroll=4)
    def j_loop(j, carry):
        fx, fy, fz, e = carry
        jf = jnp.full((16,), j, jnp.int32)
        xj = plsc.load_gather(xs_v, [jf])
        yj = plsc.load_gather(ys_v, [jf])
        zj = plsc.load_gather(zs_v, [jf])
        sj = plsc.load_gather(sp_v, [jf])
        idx = si4 + sj
        t3g = plsc.load_gather(t3_v, [idx])
        tag = plsc.load_gather(ta_v, [idx])
        teg = plsc.load_gather(te_v, [idx])
        tfg = plsc.load_gather(tf_v, [idx])
        dx = xj - xi
        dy = yj - yi
        dz = zj - zi
        d2 = dx * dx + dy * dy + dz * dz
        y0 = plsc.bitcast(
            jnp.int32(0x5F3759DF) - (plsc.bitcast(d2, jnp.int32) >> 1),
            jnp.float32)
        hd = 0.5 * d2
        y0 = y0 * (1.5 - hd * y0 * y0)
        y0 = y0 * (1.5 - hd * y0 * y0)
        r = d2 * y0
        xr = 1.0 - r * t3g
        mask = (jf != iv) & (r < _CUTOFF) & (xr > 0.0)
        xq = jnp.where(mask, xr, 1.0)
        xb = plsc.bitcast(xq, jnp.int32)
        ef = ((xb >> 23) - 127).astype(jnp.float32)
        mb = plsc.bitcast((xb & 0x007FFFFF) | 0x3F800000, jnp.float32)
        poly = jnp.full((16,), _LOG2_COEFFS[6], jnp.float32)
        for c in _LOG2_COEFFS[5::-1]:
            poly = poly * mb + c
        lg2 = poly + ef
        p1 = jnp.exp(tag * lg2)
        e = e + jnp.where(mask, teg * p1 * xq, 0.0)
        g = jnp.where(mask, (tfg * p1) * y0, 0.0)
        return fx + g * dx, fy + g * dy, fz + g * dz, e

    fx, fy, fz, e2 = j_loop
    fxv[...] = fx
    fyv[...] = fy
    fzv[...] = fz
    ev[...] = e2

    out_base = half * _SC_ROWS + (base - (_N - _SC_ROWS))
    pltpu.sync_copy(fxv, fx_h.at[pl.ds(out_base, 16)])
    pltpu.sync_copy(fyv, fy_h.at[pl.ds(out_base, 16)])
    pltpu.sync_copy(fzv, fz_h.at[pl.ds(out_base, 16)])
    pltpu.sync_copy(ev, e_h.at[pl.ds(wid * 16, 16)])


def _sc_call(xs, ys, zs, spec, t3_t, ta_t, te_t, tf_t):
    mesh = plsc.VectorSubcoreMesh(core_axis_name="c", subcore_axis_name="s")
    f32 = jnp.float32
    run = pl.kernel(
        _sc_body,
        mesh=mesh,
        compiler_params=pltpu.CompilerParams(needs_layout_passes=False,
                                             skip_device_barrier=True),
        out_type=[
            jax.ShapeDtypeStruct((_SC_SPLIT * _SC_ROWS,), f32),
            jax.ShapeDtypeStruct((_SC_SPLIT * _SC_ROWS,), f32),
            jax.ShapeDtypeStruct((_SC_SPLIT * _SC_ROWS,), f32),
            jax.ShapeDtypeStruct((_NW * 16,), f32),
        ],
        scratch_types=[
            pltpu.VMEM((_N,), f32),
            pltpu.VMEM((_N,), f32),
            pltpu.VMEM((_N,), f32),
            pltpu.VMEM((_N,), jnp.int32),
            pltpu.VMEM((16,), f32),
            pltpu.VMEM((16,), f32),
            pltpu.VMEM((16,), f32),
            pltpu.VMEM((16,), f32),
            pltpu.VMEM((16,), f32),
            pltpu.VMEM((16,), f32),
            pltpu.VMEM((16,), f32),
            pltpu.VMEM((16,), f32),
        ],
    )
    return run(xs, ys, zs, spec, t3_t, ta_t, te_t, tf_t)


def _tc_body(pos_ref, posT_ref, sc_ref, sr_ref, posf_ref, sig_ref, alp_ref,
             eps_ref, e_ref, f_ref):
    pid = pl.program_id(0)
    n = posT_ref.shape[1]
    blk = pos_ref.shape[0]

    px = posT_ref[0:1, :]
    py = posT_ref[1:2, :]
    pz = posT_ref[2:3, :]
    cx = pos_ref[:, 0:1]
    cy = pos_ref[:, 1:2]
    cz = pos_ref[:, 2:3]
    dx = px - cx
    dy = py - cy
    dz = pz - cz
    d2 = dx * dx + dy * dy + dz * dz
    rinv = jax.lax.rsqrt(d2)
    r = d2 * rinv

    oh_i = (sc_ref[...] == jax.lax.broadcasted_iota(
        jnp.int32, (blk, _NSPEC), 1)).astype(jnp.float32)
    oh_j = (sr_ref[...] == jax.lax.broadcasted_iota(
        jnp.int32, (_NSPEC, n), 0)).astype(jnp.float32)

    def pair_field(t44):
        return jnp.dot(
            jnp.dot(oh_i, t44, preferred_element_type=jnp.float32),
            oh_j, preferred_element_type=jnp.float32)

    t3 = pair_field(1.0 / sig_ref[...])
    ta = pair_field(alp_ref[...] - 1.0)
    te = pair_field(eps_ref[...] / alp_ref[...])
    tf = pair_field(-eps_ref[...] / sig_ref[...])

    xr = 1.0 - r * t3
    mask = (d2 > 0.0) & (r < _CUTOFF) & (xr > 0.0)

    xq = jnp.where(mask, xr, 1.0)
    p1 = jnp.exp(ta * jnp.log(xq))
    energies = jnp.where(mask, te * (p1 * xq), 0.0)
    g = jnp.where(mask, (tf * p1) * rinv, 0.0)

    s = jnp.sum(g, axis=1, keepdims=True)
    f_ref[...] = (jnp.dot(g, posf_ref[...], preferred_element_type=jnp.float32)
                  - s * pos_ref[...])

    @pl.when(pid == 0)
    def _():
        e_ref[...] = jnp.zeros((1, 1), jnp.float32)

    e_ref[...] += 0.5 * jnp.sum(energies, keepdims=True)


def _tc_call(positions, species):
    n = positions.shape[0]
    nrows = n - _SC_ROWS
    posT = positions.T
    spec_col = species.reshape(n, 1).astype(jnp.int32)
    spec_row = species.reshape(1, n).astype(jnp.int32)

    def run(sig_m, alp_m, eps_m):
        return pl.pallas_call(
            _tc_body,
            grid=(nrows // _TC_BLOCK,),
            compiler_params=pltpu.CompilerParams(skip_device_barrier=True),
            in_specs=[
                pl.BlockSpec((_TC_BLOCK, 3), lambda i: (i, 0)),
                pl.BlockSpec((3, n), lambda i: (0, 0)),
                pl.BlockSpec((_TC_BLOCK, 1), lambda i: (i, 0)),
                pl.BlockSpec((1, n), lambda i: (0, 0)),
                pl.BlockSpec((n, 3), lambda i: (0, 0)),
                pl.BlockSpec((_NSPEC, _NSPEC), lambda i: (0, 0)),
                pl.BlockSpec((_NSPEC, _NSPEC), lambda i: (0, 0)),
                pl.BlockSpec((_NSPEC, _NSPEC), lambda i: (0, 0)),
            ],
            out_specs=[
                pl.BlockSpec((1, 1), lambda i: (0, 0)),
                pl.BlockSpec((_TC_BLOCK, 3), lambda i: (i, 0)),
            ],
            out_shape=[
                jax.ShapeDtypeStruct((1, 1), jnp.float32),
                jax.ShapeDtypeStruct((nrows, 3), jnp.float32),
            ],
        )(positions, posT, spec_col, spec_row, positions,
          sig_m, alp_m, eps_m)

    return run


def kernel(positions, species, sigma_matrix, epsilon_matrix, alpha_matrix):
    xs = positions[:, 0]
    ys = positions[:, 1]
    zs = positions[:, 2]
    spec = species.astype(jnp.int32)
    t3_t = (1.0 / sigma_matrix).reshape(-1)
    ta_t = ((alpha_matrix - 1.0) * _LN2).reshape(-1)
    te_t = (epsilon_matrix / alpha_matrix).reshape(-1)
    tf_t = (-epsilon_matrix / sigma_matrix).reshape(-1)
    fxp, fyp, fzp, e_part = _sc_call(xs, ys, zs, spec, t3_t, ta_t, te_t, tf_t)
    fx = jnp.sum(fxp.reshape(_SC_SPLIT, _SC_ROWS), axis=0)
    fy = jnp.sum(fyp.reshape(_SC_SPLIT, _SC_ROWS), axis=0)
    fz = jnp.sum(fzp.reshape(_SC_SPLIT, _SC_ROWS), axis=0)
    e_tc, f_tc = _tc_call(positions, species)(
        sigma_matrix, alpha_matrix, epsilon_matrix)
    forces = jnp.concatenate(
        [f_tc, jnp.stack([fx, fy, fz], axis=1)], axis=0)
    energy = 0.5 * jnp.sum(e_part) + e_tc[0, 0]
    return energy, forces
